# baseline (speedup 1.0000x reference)
import functools

import jax
import jax.numpy as jnp
from jax import lax
from jax.experimental import pallas as pl
from jax.experimental.pallas import tpu as pltpu
from jax.experimental.pallas import tpu_sc as plsc

B = 16384
D = 32
PACK = 4
NLINES = 250000
NC = 2
NS = 16
NW = NC * NS
BPW = B // NW
CHUNK = 128
NCHUNK = BPW // CHUNK

EPS = 1e-5


def _gather_body(uids, mids, q_mfu, q_mfm, q_mlpu, q_mlpm,
                 o_mfu, o_mfm, o_mlpu, o_mlpm,
                 uidx_v, midx_v, line_v, big_v, rows_v, sem):
    wid = lax.axis_index("s") * NC + lax.axis_index("c")
    base = wid * BPW
    pltpu.sync_copy(uids.at[pl.ds(base, BPW)], uidx_v)
    pltpu.sync_copy(mids.at[pl.ds(base, BPW)], midx_v)

    for tbl, idx_v, out in ((q_mfu, uidx_v, o_mfu), (q_mfm, midx_v, o_mfm),
                            (q_mlpu, uidx_v, o_mlpu), (q_mlpm, midx_v, o_mlpm)):
        def linecalc(g, carry):
            line_v[pl.ds(g * 16, 16)] = jax.lax.shift_right_logical(
                idx_v[pl.ds(g * 16, 16)], 2)
            return carry
        lax.fori_loop(0, BPW // 16, linecalc, 0)

        for h in range(2):
            h0 = h * (BPW // 2)
            copies = []
            for j in range(NCHUNK // 2):
                src_sl = pl.ds(h0 + j * CHUNK, CHUNK)
                dst_sl = pl.ds(j * CHUNK, CHUNK)
                copies.append(pltpu.async_copy(tbl.at[line_v.at[src_sl]],
                                               big_v.at[dst_sl], sem))
            for c in copies:
                c.wait()

            def extract(g, carry):
                j0 = g * 16
                vq = jax.lax.shift_left(
                    idx_v[pl.ds(h0 + j0, 16)] & 3, 5)
                for l in range(16):
                    q32 = vq[l]
                    j = j0 + l
                    rows_v[h0 + j, pl.ds(0, 16)] = big_v[j, pl.ds(q32, 16)]
                    rows_v[h0 + j, pl.ds(16, 16)] = big_v[j, pl.ds(q32 + 16, 16)]
                return carry
            lax.fori_loop(0, BPW // 32, extract, 0)
        pltpu.sync_copy(rows_v, out.at[pl.ds(base, BPW)])


_gather = functools.partial(
    pl.kernel,
    out_type=[jax.ShapeDtypeStruct((B, D), jnp.float32)] * 4,
    mesh=plsc.VectorSubcoreMesh(core_axis_name="c", subcore_axis_name="s"),
    scratch_types=[
        pltpu.VMEM((BPW,), jnp.int32),
        pltpu.VMEM((BPW,), jnp.int32),
        pltpu.VMEM((BPW,), jnp.int32),
        pltpu.VMEM((BPW // 2, 4 * D), jnp.float32),
        pltpu.VMEM((BPW, D), jnp.float32),
        pltpu.SemaphoreType.DMA,
    ],
)(_gather_body)


def _dense_body(mfu, mfm, mlpu, mlpm, w1u, w1m, c1, w2, c2, w3, c3,
                wfm, wfx, bf, out):
    f32 = jnp.float32
    x1 = jnp.dot(mlpu[...], w1u[...], preferred_element_type=f32)
    x1 += jnp.dot(mlpm[...], w1m[...], preferred_element_type=f32)
    x1 = jnp.maximum(x1 + c1[...], 0.0)
    x2 = jnp.maximum(jnp.dot(x1, w2[...], preferred_element_type=f32) + c2[...], 0.0)
    x3 = jnp.maximum(jnp.dot(x2, w3[...], preferred_element_type=f32) + c3[...], 0.0)
    mf = mfu[...] * mfm[...]
    logit = jnp.dot(mf, wfm[...], preferred_element_type=f32)
    logit += jnp.dot(x3, wfx[...], preferred_element_type=f32)
    logit += bf[...]
    out[...] = jax.nn.sigmoid(logit)


def _dense(mfu, mfm, mlpu, mlpm, w1u, w1m, c1, w2, c2, w3, c3, wfm, wfx, bf):
    bs = 2048
    grid = (B // bs,)
    row_spec = pl.BlockSpec((bs, D), lambda i: (i, 0))
    full = lambda shape: pl.BlockSpec(shape, lambda i: tuple(0 for _ in shape))
    return pl.pallas_call(
        _dense_body,
        grid=grid,
        in_specs=[
            row_spec, row_spec, row_spec, row_spec,
            full((D, 64)), full((D, 64)), full((1, 64)),
            full((64, 32)), full((1, 32)),
            full((32, 16)), full((1, 16)),
            full((D, 1)), full((16, 1)), full((1, 1)),
        ],
        out_specs=pl.BlockSpec((bs, 1), lambda i: (i, 0)),
        out_shape=jax.ShapeDtypeStruct((B, 1), jnp.float32),
    )(mfu, mfm, mlpu, mlpm, w1u, w1m, c1, w2, c2, w3, c3, wfm, wfx, bf)


def kernel(user_ids, movie_ids, mf_user_emb, mf_movie_emb, mlp_user_emb,
           mlp_movie_emb, W1, b1, g1, bt1, W2, b2, g2, bt2, W3, b3, g3, bt3,
           Wf, bf):
    uids = user_ids.astype(jnp.int32)
    mids = movie_ids.astype(jnp.int32)

    packed = [jnp.reshape(t, (NLINES, PACK * D))
              for t in (mf_user_emb, mf_movie_emb, mlp_user_emb, mlp_movie_emb)]
    mfu, mfm, mlpu, mlpm = _gather(uids, mids, *packed)

    inv = 1.0 / jnp.sqrt(1.0 + EPS)
    a1 = g1 * inv
    a2 = g2 * inv
    a3 = g3 * inv
    w1f = (W1 * a1[:, None]).T
    c1 = (b1 * a1 + bt1)[None, :]
    w2f = (W2 * a2[:, None]).T
    c2 = (b2 * a2 + bt2)[None, :]
    w3f = (W3 * a3[:, None]).T
    c3 = (b3 * a3 + bt3)[None, :]
    wfm = Wf[:, :D].T
    wfx = Wf[:, D:].T
    bfr = bf[None, :]

    return _dense(mfu, mfm, mlpu, mlpm, w1f[:D], w1f[D:], c1, w2f, c2,
                  w3f, c3, wfm, wfx, bfr)
